# Initial kernel scaffold; baseline (speedup 1.0000x reference)
import jax
import jax.numpy as jnp
from jax.experimental import pallas as pl


def kernel(x, tables):
    raise NotImplementedError("write your pallas kernel here")


# speedup vs baseline: 1.2076x; 1.2076x over previous
import functools

import jax
import jax.numpy as jnp
from jax import lax
from jax.experimental import pallas as pl
from jax.experimental.pallas import tpu as pltpu
from jax.experimental.pallas import tpu_sc as plsc

NUM_FIELDS = 26
VOCAB = 100000
EMBED_DIM = 32
BATCH = 16384

_INFO = plsc.get_sparse_core_info()
_NC, _NS = _INFO.num_cores, _INFO.num_subcores
_NW = _NC * _NS
_N = BATCH * NUM_FIELDS
_PER_W = _N // _NW
_IW = 128
_ROWS_PER_W = _PER_W // _IW
_GRP = 8
_STEPS = _ROWS_PER_W // _GRP
_CHUNK = _GRP * _IW


def _make_gather():
    mesh = plsc.VectorSubcoreMesh(core_axis_name="c", subcore_axis_name="s")

    @functools.partial(
        pl.kernel,
        mesh=mesh,
        out_type=jax.ShapeDtypeStruct((_N, EMBED_DIM), jnp.float32),
        scratch_types=[
            pltpu.VMEM((_ROWS_PER_W, _IW), jnp.int32),
            pltpu.VMEM((_CHUNK, EMBED_DIM), jnp.float32),
            pltpu.SemaphoreType.DMA,
        ],
        compiler_params=pltpu.CompilerParams(use_tc_tiling_on_sc=False),
    )
    def gather_kernel(tab_hbm, idx_hbm, out_hbm, idx_v, rows_v, sem):
        wid = lax.axis_index("s") * _NC + lax.axis_index("c")
        pltpu.sync_copy(idx_hbm.at[pl.ds(wid * _ROWS_PER_W, _ROWS_PER_W)], idx_v)
        out_base = wid * _PER_W

        def step(c, _):
            handles = []
            for j in range(_GRP):
                h = pltpu.async_copy(
                    tab_hbm.at[idx_v.at[c * _GRP + j]],
                    rows_v.at[pl.ds(j * _IW, _IW)],
                    sem,
                )
                handles.append(h)
            for h in handles:
                h.wait()
            pltpu.sync_copy(
                rows_v, out_hbm.at[pl.ds(out_base + c * _CHUNK, _CHUNK)]
            )
            return ()

        lax.fori_loop(0, _STEPS, step, (), unroll=False)

    return gather_kernel


_gather = _make_gather()


def kernel(x, tables):
    tab_flat = tables.reshape(NUM_FIELDS * VOCAB, EMBED_DIM)
    offs = (jnp.arange(NUM_FIELDS, dtype=jnp.int32) * VOCAB)[None, :]
    idx = (x.astype(jnp.int32) + offs).reshape(_N // _IW, _IW)
    out = _gather(tab_flat, idx)
    return out.reshape(BATCH, NUM_FIELDS * EMBED_DIM)


# speedup vs baseline: 1.2148x; 1.0060x over previous
import functools

import jax
import jax.numpy as jnp
from jax import lax
from jax.experimental import pallas as pl
from jax.experimental.pallas import tpu as pltpu
from jax.experimental.pallas import tpu_sc as plsc

NUM_FIELDS = 26
VOCAB = 100000
EMBED_DIM = 32
BATCH = 16384

_INFO = plsc.get_sparse_core_info()
_NC, _NS = _INFO.num_cores, _INFO.num_subcores
_NW = _NC * _NS
_N = BATCH * NUM_FIELDS
_PER_W = _N // _NW
_IW = 128
_ROWS_PER_W = _PER_W // _IW
_GRP = 13
_STEPS = _ROWS_PER_W // _GRP
_CHUNK = _GRP * _IW


def _make_gather():
    mesh = plsc.VectorSubcoreMesh(core_axis_name="c", subcore_axis_name="s")

    @functools.partial(
        pl.kernel,
        mesh=mesh,
        out_type=jax.ShapeDtypeStruct((_N, EMBED_DIM), jnp.float32),
        scratch_types=[
            pltpu.VMEM((_ROWS_PER_W, _IW), jnp.int32),
            pltpu.VMEM((_CHUNK, EMBED_DIM), jnp.float32),
            pltpu.VMEM((_CHUNK, EMBED_DIM), jnp.float32),
            pltpu.SemaphoreType.DMA,
            pltpu.SemaphoreType.DMA,
            pltpu.SemaphoreType.DMA,
            pltpu.SemaphoreType.DMA,
        ],
        compiler_params=pltpu.CompilerParams(use_tc_tiling_on_sc=False),
    )
    def gather_kernel(tab_hbm, idx_hbm, out_hbm, idx_v, rows0, rows1,
                      sg0, sg1, sw0, sw1):
        wid = lax.axis_index("s") * _NC + lax.axis_index("c")
        pltpu.sync_copy(idx_hbm.at[pl.ds(wid * _ROWS_PER_W, _ROWS_PER_W)], idx_v)
        out_base = wid * _PER_W

        def fire(c, buf, sem):
            for j in range(_GRP):
                pltpu.async_copy(
                    tab_hbm.at[idx_v.at[c * _GRP + j]],
                    buf.at[pl.ds(j * _IW, _IW)],
                    sem,
                )

        def drain(buf, sem):
            pltpu.make_async_copy(out_hbm.at[pl.ds(0, _CHUNK)], buf, sem).wait()

        def write(c, buf, sem):
            pltpu.async_copy(
                buf, out_hbm.at[pl.ds(out_base + c * _CHUNK, _CHUNK)], sem
            )

        fire(0, rows0, sg0)

        def pair(k, _):
            c0 = 2 * k

            @pl.when(k >= 1)
            def _():
                drain(rows1, sw1)
            fire(c0 + 1, rows1, sg1)
            drain(rows0, sg0)
            write(c0, rows0, sw0)

            @pl.when(c0 + 2 < _STEPS)
            def _():
                drain(rows0, sw0)
                fire(c0 + 2, rows0, sg0)
            drain(rows1, sg1)
            write(c0 + 1, rows1, sw1)
            return ()

        lax.fori_loop(0, _STEPS // 2, pair, (), unroll=False)
        drain(rows0, sw0)
        drain(rows1, sw1)

    return gather_kernel


_gather = _make_gather()


def kernel(x, tables):
    tab_flat = tables.reshape(NUM_FIELDS * VOCAB, EMBED_DIM)
    offs = (jnp.arange(NUM_FIELDS, dtype=jnp.int32) * VOCAB)[None, :]
    idx = (x.astype(jnp.int32) + offs).reshape(_N // _IW, _IW)
    out = _gather(tab_flat, idx)
    return out.reshape(BATCH, NUM_FIELDS * EMBED_DIM)
